# baseline (speedup 1.0000x reference)
import functools

import jax
import jax.numpy as jnp
from jax.experimental import pallas as pl
from jax.experimental.pallas import tpu as pltpu

NODE_D = 128
EDGE_D = 16
HID = 128


def _proj_body(nodes_ref, wr_ref, ws_ref, p_ref, q_ref):
    x = nodes_ref[...]
    p_ref[...] = jnp.dot(x, wr_ref[...], preferred_element_type=jnp.float32)
    q_ref[...] = jnp.dot(x, ws_ref[...], preferred_element_type=jnp.float32)


def _node_proj(flat_nodes, w_recv, w_send):
    bn = flat_nodes.shape[0]
    blk = 2000
    grid = (bn // blk,)
    return pl.pallas_call(
        _proj_body,
        grid=grid,
        in_specs=[
            pl.BlockSpec((blk, NODE_D), lambda i: (i, 0)),
            pl.BlockSpec((NODE_D, HID), lambda i: (0, 0)),
            pl.BlockSpec((NODE_D, HID), lambda i: (0, 0)),
        ],
        out_specs=[
            pl.BlockSpec((blk, HID), lambda i: (i, 0)),
            pl.BlockSpec((blk, HID), lambda i: (i, 0)),
        ],
        out_shape=[
            jax.ShapeDtypeStruct((bn, HID), jnp.float32),
            jax.ShapeDtypeStruct((bn, HID), jnp.float32),
        ],
    )(flat_nodes, w_recv, w_send)


def _edge_body(g_ref, e_ref, we_ref, bin_ref, wout_ref, bout_ref,
               wedge_ref, bedge_ref, ne_ref, eo_ref):
    h = g_ref[...] + jnp.dot(e_ref[...], we_ref[...],
                             preferred_element_type=jnp.float32) + bin_ref[...]
    h = jnp.maximum(h, 0.0)
    ne = jnp.dot(h, wout_ref[...], preferred_element_type=jnp.float32) + bout_ref[...]
    ne = jnp.maximum(ne, 0.0)
    ne_ref[...] = ne
    eo = jnp.dot(ne, wedge_ref[...], preferred_element_type=jnp.float32) + bedge_ref[...]
    eo_ref[...] = jnp.maximum(eo, 0.0)


def _edge_dense(g, flat_edges, w_edge_in, b_in, w_out, b_out, w_edge, b_edge):
    be = g.shape[0]
    blk = 2000
    grid = (be // blk,)
    return pl.pallas_call(
        _edge_body,
        grid=grid,
        in_specs=[
            pl.BlockSpec((blk, HID), lambda i: (i, 0)),
            pl.BlockSpec((blk, EDGE_D), lambda i: (i, 0)),
            pl.BlockSpec((EDGE_D, HID), lambda i: (0, 0)),
            pl.BlockSpec((1, HID), lambda i: (0, 0)),
            pl.BlockSpec((HID, NODE_D), lambda i: (0, 0)),
            pl.BlockSpec((1, NODE_D), lambda i: (0, 0)),
            pl.BlockSpec((HID, EDGE_D), lambda i: (0, 0)),
            pl.BlockSpec((1, EDGE_D), lambda i: (0, 0)),
        ],
        out_specs=[
            pl.BlockSpec((blk, NODE_D), lambda i: (i, 0)),
            pl.BlockSpec((blk, EDGE_D), lambda i: (i, 0)),
        ],
        out_shape=[
            jax.ShapeDtypeStruct((be, NODE_D), jnp.float32),
            jax.ShapeDtypeStruct((be, EDGE_D), jnp.float32),
        ],
    )(g, flat_edges, w_edge_in, b_in[None, :], w_out, b_out[None, :],
      w_edge, b_edge[None, :])


def kernel(nodes, edges, senders, receivers, W_in, b_in, W_out, b_out, W_edge, b_edge):
    b, n, d_node = nodes.shape
    e = edges.shape[1]
    d_edge = edges.shape[2]

    s = jnp.reshape(senders, (b * e,))
    r = jnp.reshape(receivers, (b * e,))
    offsets = jnp.reshape(
        jnp.tile((jnp.arange(b, dtype=s.dtype) * n)[:, None], (1, e)), (-1,))
    s_off = jnp.where(s != -1, s + offsets, s)
    r_off = jnp.where(r != -1, r + offsets, r)

    flat_nodes = jnp.reshape(nodes, (b * n, d_node))
    flat_edges = jnp.reshape(edges, (b * e, d_edge))

    w_recv = W_in[:NODE_D]
    w_edge_in = W_in[NODE_D:NODE_D + EDGE_D]
    w_send = W_in[NODE_D + EDGE_D:]

    p, q = _node_proj(flat_nodes, w_recv, w_send)
    g = jnp.take(p, r_off, axis=0) + jnp.take(q, s_off, axis=0)

    new_edges, edges_out = _edge_dense(
        g, flat_edges, w_edge_in, b_in, W_out, b_out, W_edge, b_edge)

    sums = jax.ops.segment_sum(new_edges, r_off, num_segments=b * n)
    cnt = jax.ops.segment_sum(jnp.ones((b * e,), jnp.float32), r_off,
                              num_segments=b * n)
    new_nodes = sums / jnp.maximum(cnt, 1.0)[:, None]

    return (jnp.reshape(new_nodes, (b, n, d_node)),
            jnp.reshape(edges_out, (b, e, d_edge)),
            s, r)


# speedup vs baseline: 1.7448x; 1.7448x over previous
import functools

import jax
import jax.numpy as jnp
from jax import lax
from jax.experimental import pallas as pl
from jax.experimental.pallas import tpu as pltpu
from jax.experimental.pallas import tpu_sc as plsc

NODE_D = 128
EDGE_D = 16
HID = 128

SC_CORES = 2
SC_SUBCORES = 16
SC_WORKERS = SC_CORES * SC_SUBCORES
G_CHUNK = 400


def _sc_gather_body(p_hbm, q_hbm, ridx_hbm, sidx_hbm, g_hbm,
                    idx_r, idx_s, rows_a, rows_b, sem_a, sem_b):
    wid = lax.axis_index("s") * SC_CORES + lax.axis_index("c")
    edges_per_w = g_hbm.shape[0] // SC_WORKERS
    nchunk = edges_per_w // G_CHUNK
    base = wid * edges_per_w

    def chunk(i, carry):
        off = base + i * G_CHUNK
        pltpu.sync_copy(ridx_hbm.at[pl.ds(off, G_CHUNK)], idx_r)
        cp_a = pltpu.async_copy(p_hbm.at[idx_r], rows_a, sem_a)
        pltpu.sync_copy(sidx_hbm.at[pl.ds(off, G_CHUNK)], idx_s)
        cp_b = pltpu.async_copy(q_hbm.at[idx_s], rows_b, sem_b)
        cp_a.wait()
        cp_b.wait()

        def addrow(j, c):
            for k in range(HID // 16):
                sl = pl.ds(k * 16, 16)
                rows_a[j, sl] = rows_a[j, sl] + rows_b[j, sl]
            return c

        lax.fori_loop(0, G_CHUNK, addrow, 0)
        pltpu.sync_copy(rows_a, g_hbm.at[pl.ds(off, G_CHUNK)])
        return carry

    lax.fori_loop(0, nchunk, chunk, 0)


def _sc_gather(p, q, r_off, s_off):
    be = r_off.shape[0]
    mesh = plsc.VectorSubcoreMesh(core_axis_name="c", subcore_axis_name="s",
                                  num_cores=SC_CORES, num_subcores=SC_SUBCORES)
    f = pl.kernel(
        _sc_gather_body,
        out_type=jax.ShapeDtypeStruct((be, HID), jnp.float32),
        mesh=mesh,
        scratch_types=[
            pltpu.VMEM((G_CHUNK,), jnp.int32),
            pltpu.VMEM((G_CHUNK,), jnp.int32),
            pltpu.VMEM((G_CHUNK, HID), jnp.float32),
            pltpu.VMEM((G_CHUNK, HID), jnp.float32),
            pltpu.SemaphoreType.DMA,
            pltpu.SemaphoreType.DMA,
        ],
    )
    return f(p, q, r_off, s_off)


def _proj_body(nodes_ref, wr_ref, ws_ref, p_ref, q_ref):
    x = nodes_ref[...]
    p_ref[...] = jnp.dot(x, wr_ref[...], preferred_element_type=jnp.float32)
    q_ref[...] = jnp.dot(x, ws_ref[...], preferred_element_type=jnp.float32)


def _node_proj(flat_nodes, w_recv, w_send):
    bn = flat_nodes.shape[0]
    blk = 2000
    grid = (bn // blk,)
    return pl.pallas_call(
        _proj_body,
        grid=grid,
        in_specs=[
            pl.BlockSpec((blk, NODE_D), lambda i: (i, 0)),
            pl.BlockSpec((NODE_D, HID), lambda i: (0, 0)),
            pl.BlockSpec((NODE_D, HID), lambda i: (0, 0)),
        ],
        out_specs=[
            pl.BlockSpec((blk, HID), lambda i: (i, 0)),
            pl.BlockSpec((blk, HID), lambda i: (i, 0)),
        ],
        out_shape=[
            jax.ShapeDtypeStruct((bn, HID), jnp.float32),
            jax.ShapeDtypeStruct((bn, HID), jnp.float32),
        ],
    )(flat_nodes, w_recv, w_send)


def _edge_body(g_ref, e_ref, we_ref, bin_ref, wout_ref, bout_ref,
               wedge_ref, bedge_ref, ne_ref, eo_ref):
    h = g_ref[...] + jnp.dot(e_ref[...], we_ref[...],
                             preferred_element_type=jnp.float32) + bin_ref[...]
    h = jnp.maximum(h, 0.0)
    ne = jnp.dot(h, wout_ref[...], preferred_element_type=jnp.float32) + bout_ref[...]
    ne = jnp.maximum(ne, 0.0)
    ne_ref[...] = ne
    eo = jnp.dot(ne, wedge_ref[...], preferred_element_type=jnp.float32) + bedge_ref[...]
    eo_ref[...] = jnp.maximum(eo, 0.0)


def _edge_dense(g, flat_edges, w_edge_in, b_in, w_out, b_out, w_edge, b_edge):
    be = g.shape[0]
    blk = 2000
    grid = (be // blk,)
    return pl.pallas_call(
        _edge_body,
        grid=grid,
        in_specs=[
            pl.BlockSpec((blk, HID), lambda i: (i, 0)),
            pl.BlockSpec((blk, EDGE_D), lambda i: (i, 0)),
            pl.BlockSpec((EDGE_D, HID), lambda i: (0, 0)),
            pl.BlockSpec((1, HID), lambda i: (0, 0)),
            pl.BlockSpec((HID, NODE_D), lambda i: (0, 0)),
            pl.BlockSpec((1, NODE_D), lambda i: (0, 0)),
            pl.BlockSpec((HID, EDGE_D), lambda i: (0, 0)),
            pl.BlockSpec((1, EDGE_D), lambda i: (0, 0)),
        ],
        out_specs=[
            pl.BlockSpec((blk, NODE_D), lambda i: (i, 0)),
            pl.BlockSpec((blk, EDGE_D), lambda i: (i, 0)),
        ],
        out_shape=[
            jax.ShapeDtypeStruct((be, NODE_D), jnp.float32),
            jax.ShapeDtypeStruct((be, EDGE_D), jnp.float32),
        ],
    )(g, flat_edges, w_edge_in, b_in[None, :], w_out, b_out[None, :],
      w_edge, b_edge[None, :])


def kernel(nodes, edges, senders, receivers, W_in, b_in, W_out, b_out, W_edge, b_edge):
    b, n, d_node = nodes.shape
    e = edges.shape[1]
    d_edge = edges.shape[2]

    s = jnp.reshape(senders, (b * e,))
    r = jnp.reshape(receivers, (b * e,))
    offsets = jnp.reshape(
        jnp.tile((jnp.arange(b, dtype=s.dtype) * n)[:, None], (1, e)), (-1,))
    s_off = jnp.where(s != -1, s + offsets, s)
    r_off = jnp.where(r != -1, r + offsets, r)

    flat_nodes = jnp.reshape(nodes, (b * n, d_node))
    flat_edges = jnp.reshape(edges, (b * e, d_edge))

    w_recv = W_in[:NODE_D]
    w_edge_in = W_in[NODE_D:NODE_D + EDGE_D]
    w_send = W_in[NODE_D + EDGE_D:]

    p, q = _node_proj(flat_nodes, w_recv, w_send)
    g = _sc_gather(p, q, r_off, s_off)

    new_edges, edges_out = _edge_dense(
        g, flat_edges, w_edge_in, b_in, W_out, b_out, W_edge, b_edge)

    sums = jax.ops.segment_sum(new_edges, r_off, num_segments=b * n)
    cnt = jax.ops.segment_sum(jnp.ones((b * e,), jnp.float32), r_off,
                              num_segments=b * n)
    new_nodes = sums / jnp.maximum(cnt, 1.0)[:, None]

    return (jnp.reshape(new_nodes, (b, n, d_node)),
            jnp.reshape(edges_out, (b, e, d_edge)),
            s, r)


# speedup vs baseline: 2.4840x; 1.4237x over previous
import functools

import jax
import jax.numpy as jnp
from jax import lax
from jax.experimental import pallas as pl
from jax.experimental.pallas import tpu as pltpu
from jax.experimental.pallas import tpu_sc as plsc

NODE_D = 128
EDGE_D = 16
HID = 128

SC_CORES = 2
SC_SUBCORES = 16
SC_WORKERS = SC_CORES * SC_SUBCORES
G_CHUNK = 400


def _sc_gather_body(p_hbm, q_hbm, ridx_hbm, sidx_hbm, g_hbm,
                    idx_r, idx_s, rows_a, rows_b, sem_a, sem_b):
    wid = lax.axis_index("s") * SC_CORES + lax.axis_index("c")
    edges_per_w = g_hbm.shape[0] // SC_WORKERS
    nchunk = edges_per_w // G_CHUNK
    base = wid * edges_per_w

    def chunk(i, carry):
        off = base + i * G_CHUNK
        pltpu.sync_copy(ridx_hbm.at[pl.ds(off, G_CHUNK)], idx_r)
        cp_a = pltpu.async_copy(p_hbm.at[idx_r], rows_a, sem_a)
        pltpu.sync_copy(sidx_hbm.at[pl.ds(off, G_CHUNK)], idx_s)
        cp_b = pltpu.async_copy(q_hbm.at[idx_s], rows_b, sem_b)
        cp_a.wait()
        cp_b.wait()

        def addrow(j, c):
            for k in range(HID // 16):
                sl = pl.ds(k * 16, 16)
                rows_a[j, sl] = rows_a[j, sl] + rows_b[j, sl]
            return c

        lax.fori_loop(0, G_CHUNK, addrow, 0)
        pltpu.sync_copy(rows_a, g_hbm.at[pl.ds(off, G_CHUNK)])
        return carry

    lax.fori_loop(0, nchunk, chunk, 0)


def _sc_gather(p, q, r_off, s_off):
    be = r_off.shape[0]
    mesh = plsc.VectorSubcoreMesh(core_axis_name="c", subcore_axis_name="s",
                                  num_cores=SC_CORES, num_subcores=SC_SUBCORES)
    f = pl.kernel(
        _sc_gather_body,
        out_type=jax.ShapeDtypeStruct((be, HID), jnp.float32),
        mesh=mesh,
        scratch_types=[
            pltpu.VMEM((G_CHUNK,), jnp.int32),
            pltpu.VMEM((G_CHUNK,), jnp.int32),
            pltpu.VMEM((G_CHUNK, HID), jnp.float32),
            pltpu.VMEM((G_CHUNK, HID), jnp.float32),
            pltpu.SemaphoreType.DMA,
            pltpu.SemaphoreType.DMA,
        ],
    )
    return f(p, q, r_off, s_off)


def _proj_body(nodes_ref, wr_ref, ws_ref, p_ref, q_ref):
    x = nodes_ref[...]
    p_ref[...] = jnp.dot(x, wr_ref[...], preferred_element_type=jnp.float32)
    q_ref[...] = jnp.dot(x, ws_ref[...], preferred_element_type=jnp.float32)


def _node_proj(flat_nodes, w_recv, w_send):
    bn = flat_nodes.shape[0]
    blk = 2000
    grid = (bn // blk,)
    return pl.pallas_call(
        _proj_body,
        grid=grid,
        in_specs=[
            pl.BlockSpec((blk, NODE_D), lambda i: (i, 0)),
            pl.BlockSpec((NODE_D, HID), lambda i: (0, 0)),
            pl.BlockSpec((NODE_D, HID), lambda i: (0, 0)),
        ],
        out_specs=[
            pl.BlockSpec((blk, HID), lambda i: (i, 0)),
            pl.BlockSpec((blk, HID), lambda i: (i, 0)),
        ],
        out_shape=[
            jax.ShapeDtypeStruct((bn, HID), jnp.float32),
            jax.ShapeDtypeStruct((bn, HID), jnp.float32),
        ],
    )(flat_nodes, w_recv, w_send)


def _edge_body(g_ref, e_ref, we_ref, bin_ref, wout_ref, bout_ref,
               wedge_ref, bedge_ref, ne_ref, eo_ref):
    h = g_ref[...] + jnp.dot(e_ref[...], we_ref[...],
                             preferred_element_type=jnp.float32) + bin_ref[...]
    h = jnp.maximum(h, 0.0)
    ne = jnp.dot(h, wout_ref[...], preferred_element_type=jnp.float32) + bout_ref[...]
    ne = jnp.maximum(ne, 0.0)
    ne_ref[...] = ne
    eo = jnp.dot(ne, wedge_ref[...], preferred_element_type=jnp.float32) + bedge_ref[...]
    eo_ref[...] = jnp.maximum(eo, 0.0)


def _edge_dense(g, flat_edges, w_edge_in, b_in, w_out, b_out, w_edge, b_edge):
    be = g.shape[0]
    blk = 2000
    grid = (be // blk,)
    return pl.pallas_call(
        _edge_body,
        grid=grid,
        in_specs=[
            pl.BlockSpec((blk, HID), lambda i: (i, 0)),
            pl.BlockSpec((blk, EDGE_D), lambda i: (i, 0)),
            pl.BlockSpec((EDGE_D, HID), lambda i: (0, 0)),
            pl.BlockSpec((1, HID), lambda i: (0, 0)),
            pl.BlockSpec((HID, NODE_D), lambda i: (0, 0)),
            pl.BlockSpec((1, NODE_D), lambda i: (0, 0)),
            pl.BlockSpec((HID, EDGE_D), lambda i: (0, 0)),
            pl.BlockSpec((1, EDGE_D), lambda i: (0, 0)),
        ],
        out_specs=[
            pl.BlockSpec((blk, NODE_D), lambda i: (i, 0)),
            pl.BlockSpec((blk, EDGE_D), lambda i: (i, 0)),
        ],
        out_shape=[
            jax.ShapeDtypeStruct((be, NODE_D), jnp.float32),
            jax.ShapeDtypeStruct((be, EDGE_D), jnp.float32),
        ],
    )(g, flat_edges, w_edge_in, b_in[None, :], w_out, b_out[None, :],
      w_edge, b_edge[None, :])


D_CHUNK = 160
D_HALF = 80
TRASH = 96
N_HALF = 10000
TBL = N_HALF + 240
ZSTRIPE = TBL // SC_SUBCORES
DRAIN_BLK = 400


def _sc_segsum_body(ne_hbm, lidx_hbm, seq_hbm, z_hbm, sums_hbm,
                    idxA, idxB, neA, neB, z128, gsem, table_sh):
    c = lax.axis_index("c")
    t = lax.axis_index("s")
    be = ne_hbm.shape[0]
    edges_per_t = be // SC_SUBCORES
    nchunk = edges_per_t // D_CHUNK
    nblocks = N_HALF // DRAIN_BLK
    zbase = t * ZSTRIPE
    cbase = c * be
    nbase = c * N_HALF

    pltpu.sync_copy(z_hbm, z128)

    def zchunk(k, carry):
        pltpu.sync_copy(seq_hbm.at[pl.ds(zbase + k * D_HALF, D_HALF)], idxA)
        pltpu.sync_copy(z128, table_sh.at[idxA])
        return carry
    lax.fori_loop(0, ZSTRIPE // D_HALF, zchunk, 0)
    plsc.subcore_barrier()

    def chunk(i, carry):
        ii = lax.rem(i + c * (nchunk // 2), nchunk)
        off = pl.multiple_of(t * edges_per_t + ii * D_CHUNK, 8)
        pltpu.sync_copy(lidx_hbm.at[pl.ds(cbase + off, D_HALF)], idxA)
        pltpu.sync_copy(lidx_hbm.at[pl.ds(cbase + off + D_HALF, D_HALF)], idxB)
        pltpu.sync_copy(ne_hbm.at[pl.ds(off, D_HALF)], neA)
        pltpu.sync_copy(ne_hbm.at[pl.ds(off + D_HALF, D_HALF)], neB)
        pltpu.sync_copy(neA, table_sh.at[idxA], add=True)
        pltpu.sync_copy(neB, table_sh.at[idxB], add=True)
        return carry
    lax.fori_loop(0, nchunk, chunk, 0)
    plsc.subcore_barrier()

    for dd in range(2):
        d = dd * SC_SUBCORES + t

        @pl.when(d < nblocks)
        def _():
            def sub(k, carry):
                s0 = pl.multiple_of(d * DRAIN_BLK + k * D_HALF, 8)
                pltpu.sync_copy(seq_hbm.at[pl.ds(s0, D_HALF)], idxA)
                pltpu.async_copy(table_sh.at[idxA], neA, gsem).wait()
                pltpu.sync_copy(neA, sums_hbm.at[pl.ds(nbase + s0, D_HALF)])
                return carry
            lax.fori_loop(0, DRAIN_BLK // D_HALF, sub, 0)


def _sc_segsum(ne, lidx, seq, zeros, bn):
    mesh = plsc.VectorSubcoreMesh(core_axis_name="c", subcore_axis_name="s",
                                  num_cores=SC_CORES, num_subcores=SC_SUBCORES)
    f = pl.kernel(
        _sc_segsum_body,
        out_type=jax.ShapeDtypeStruct((bn, HID), jnp.float32),
        mesh=mesh,
        scratch_types=[
            pltpu.VMEM((D_HALF,), jnp.int32),
            pltpu.VMEM((D_HALF,), jnp.int32),
            pltpu.VMEM((D_HALF, HID), jnp.float32),
            pltpu.VMEM((D_HALF, HID), jnp.float32),
            pltpu.VMEM((D_HALF, HID), jnp.float32),
            pltpu.SemaphoreType.DMA,
            pltpu.VMEM_SHARED((TBL, HID), jnp.float32),
        ],
    )
    return f(ne, lidx, seq, zeros)


def _mean_body(s_ref, c_ref, o_ref):
    o_ref[...] = s_ref[...] / jnp.maximum(c_ref[...][:, :1], 1.0)


def _mean_merge(sums, cnt8):
    bn = sums.shape[0]
    blk = 2000
    return pl.pallas_call(
        _mean_body,
        grid=(bn // blk,),
        in_specs=[
            pl.BlockSpec((blk, HID), lambda i: (i, 0)),
            pl.BlockSpec((blk, 8), lambda i: (i, 0)),
        ],
        out_specs=pl.BlockSpec((blk, HID), lambda i: (i, 0)),
        out_shape=jax.ShapeDtypeStruct((bn, HID), jnp.float32),
    )(sums, cnt8)


def kernel(nodes, edges, senders, receivers, W_in, b_in, W_out, b_out, W_edge, b_edge):
    b, n, d_node = nodes.shape
    e = edges.shape[1]
    d_edge = edges.shape[2]

    s = jnp.reshape(senders, (b * e,))
    r = jnp.reshape(receivers, (b * e,))
    offsets = jnp.reshape(
        jnp.tile((jnp.arange(b, dtype=s.dtype) * n)[:, None], (1, e)), (-1,))
    s_off = jnp.where(s != -1, s + offsets, s)
    r_off = jnp.where(r != -1, r + offsets, r)

    flat_nodes = jnp.reshape(nodes, (b * n, d_node))
    flat_edges = jnp.reshape(edges, (b * e, d_edge))

    w_recv = W_in[:NODE_D]
    w_edge_in = W_in[NODE_D:NODE_D + EDGE_D]
    w_send = W_in[NODE_D + EDGE_D:]

    p, q = _node_proj(flat_nodes, w_recv, w_send)
    g = _sc_gather(p, q, r_off, s_off)

    new_edges, edges_out = _edge_dense(
        g, flat_edges, w_edge_in, b_in, W_out, b_out, W_edge, b_edge)

    eidx = jnp.arange(b * e, dtype=jnp.int32)
    trash = N_HALF + (eidx % TRASH)
    parts = []
    for cc in range(2):
        lv = r_off - cc * N_HALF
        inb = (lv >= 0) & (lv < N_HALF)
        parts.append(jnp.where(inb, lv, trash))
    lidx = jnp.concatenate(parts)
    seq = jnp.arange(TBL, dtype=jnp.int32)
    zeros = jnp.zeros((D_HALF, HID), jnp.float32)

    sums = _sc_segsum(new_edges, lidx, seq, zeros, b * n)
    cnt = jax.ops.segment_sum(jnp.ones((b * e,), jnp.float32), r_off,
                              num_segments=b * n)
    cnt8 = jnp.broadcast_to(cnt[:, None], (b * n, 8))
    new_nodes = _mean_merge(sums, cnt8)

    return (jnp.reshape(new_nodes, (b, n, d_node)),
            jnp.reshape(edges_out, (b, e, d_edge)),
            s, r)
